# baseline (speedup 1.0000x reference)
import functools

import jax
import jax.numpy as jnp
from jax import lax
from jax.experimental import pallas as pl
from jax.experimental.pallas import tpu as pltpu
from jax.experimental.pallas import tpu_sc as plsc

_DIM = 1024
_SCALE = _DIM ** (-0.5)
_NC, _NS, _L = 2, 16, 16
_NW = _NC * _NS
_CH = 32
_NBUF = 3


def _sc_scale(emb):
    rows = emb.shape[0]
    rows_w = rows // _NW
    nch = rows_w // _CH
    vecs = _CH * (_DIM // _L)

    mesh = plsc.VectorSubcoreMesh(core_axis_name="c", subcore_axis_name="s")

    @functools.partial(
        pl.kernel,
        out_type=jax.ShapeDtypeStruct((rows, _DIM), jnp.float32),
        mesh=mesh,
        scratch_types=[
            pltpu.VMEM((_CH, _DIM), jnp.float32),
            pltpu.VMEM((_CH, _DIM), jnp.float32),
            pltpu.VMEM((_CH, _DIM), jnp.float32),
            pltpu.SemaphoreType.DMA,
            pltpu.SemaphoreType.DMA,
            pltpu.SemaphoreType.DMA,
            pltpu.SemaphoreType.DMA,
            pltpu.SemaphoreType.DMA,
            pltpu.SemaphoreType.DMA,
        ],
    )
    def k(emb_hbm, out_hbm, buf0, buf1, buf2, si0, si1, si2, so0, so1, so2):
        wid = lax.axis_index("s") * _NC + lax.axis_index("c")
        base = wid * rows_w
        bufs = (buf0, buf1, buf2)
        sin = (si0, si1, si2)
        sout = (so0, so1, so2)

        def in_copy(ch):
            b = ch % _NBUF
            return pltpu.async_copy(
                emb_hbm.at[pl.ds(base + ch * _CH, _CH)], bufs[b], sin[b])

        def out_copy(ch):
            b = ch % _NBUF
            return pltpu.async_copy(
                bufs[b], out_hbm.at[pl.ds(base + ch * _CH, _CH)], sout[b])

        in_d = {ch: in_copy(ch) for ch in range(min(_NBUF, nch))}
        out_d = {}
        for ch in range(nch):
            if ch >= _NBUF - 1 and ch + 1 < nch:
                out_d[ch - (_NBUF - 1)].wait()
                in_d[ch + 1] = in_copy(ch + 1)
            in_d[ch].wait()
            buf = bufs[ch % _NBUF]

            @plsc.parallel_loop(0, vecs, unroll=8)
            def _body(j):
                r = j >> 6
                c = pl.multiple_of((j & 63) << 4, _L)
                buf[r, pl.ds(c, _L)] = buf[r, pl.ds(c, _L)] * _SCALE

            out_d[ch] = out_copy(ch)

        for ch in range(max(0, nch - _NBUF), nch):
            out_d[ch].wait()

    return k(emb)


def kernel(x, emb):
    seq_len = x.shape[1]
    return _sc_scale(emb[:seq_len])
